# Initial kernel scaffold; baseline (speedup 1.0000x reference)
import jax
import jax.numpy as jnp
from jax.experimental import pallas as pl


def kernel(idx, targets, table):
    raise NotImplementedError("write your pallas kernel here")


# speedup vs baseline: 3.6087x; 3.6087x over previous
import functools

import jax
import jax.numpy as jnp
from jax import lax
from jax.experimental import pallas as pl
from jax.experimental.pallas import tpu as pltpu
from jax.experimental.pallas import tpu_sc as plsc

N = 204800
C = 128
NC = 2
NS = 16
NW = NC * NS
ROWS_PER_W = N // NW
CH = 128
NCH = ROWS_PER_W // CH

@functools.cache
def _make_sc_gather():
    mesh = plsc.VectorSubcoreMesh(core_axis_name="c", subcore_axis_name="s")

    @functools.partial(
        pl.kernel,
        mesh=mesh,
        out_type=jax.ShapeDtypeStruct((N, C), jnp.float32),
        scratch_types=[
            pltpu.VMEM((CH,), jnp.int32),
            pltpu.VMEM((CH, C), jnp.float32),
            pltpu.SemaphoreType.DMA,
        ],
    )
    def _sc_gather(idx_hbm, table_hbm, out_hbm, idx_v, rows_v, sem):
        wid = lax.axis_index("s") * NC + lax.axis_index("c")
        base = wid * ROWS_PER_W

        def step(i, carry):
            off = base + i * CH
            pltpu.sync_copy(idx_hbm.at[pl.ds(off, CH)], idx_v)
            pltpu.async_copy(table_hbm.at[idx_v], rows_v, sem).wait()
            pltpu.sync_copy(rows_v, out_hbm.at[pl.ds(off, CH)])
            return carry

        lax.fori_loop(0, NCH, step, 0)

    return _sc_gather


RB = 2048
G = N // RB


def _loss_body(x_ref, t_ref, out_ref):
    x = x_ref[...]
    t = t_ref[0, 0]
    m = jnp.max(x, axis=-1)
    s = jnp.sum(jnp.exp(x - m[:, None]), axis=-1)
    lse = m + jnp.log(s)
    cls = lax.broadcasted_iota(jnp.int32, (RB, C), 1)
    picked = jnp.sum(jnp.where(cls == t[:, None], x, 0.0), axis=-1)
    blk = jnp.sum(lse - picked)

    @pl.when(pl.program_id(0) == 0)
    def _():
        out_ref[0, 0] = 0.0

    out_ref[0, 0] += blk

    @pl.when(pl.program_id(0) == G - 1)
    def _():
        out_ref[0, 0] = out_ref[0, 0] / N


_tc_loss = pl.pallas_call(
    _loss_body,
    grid=(G,),
    in_specs=[
        pl.BlockSpec((RB, C), lambda i: (i, 0)),
        pl.BlockSpec((1, 1, RB), lambda i: (i, 0, 0)),
    ],
    out_specs=pl.BlockSpec((1, 1), lambda i: (0, 0), memory_space=pltpu.SMEM),
    out_shape=jax.ShapeDtypeStruct((1, 1), jnp.float32),
)


def kernel(idx, targets, table):
    idx_flat = idx.reshape(-1).astype(jnp.int32)
    logits = _make_sc_gather()(idx_flat, table)
    tgt = targets.reshape(G, 1, RB).astype(jnp.int32)
    loss = _tc_loss(logits, tgt)[0, 0]
    return logits, loss


# speedup vs baseline: 4.5839x; 1.2702x over previous
import functools

import jax
import jax.numpy as jnp
from jax import lax
from jax.experimental import pallas as pl
from jax.experimental.pallas import tpu as pltpu
from jax.experimental.pallas import tpu_sc as plsc

N = 204800
C = 128
NC = 2
NS = 16
NW = NC * NS
ROWS_PER_W = N // NW
CH = 128
NCH = ROWS_PER_W // CH

NBUF = 5


@functools.cache
def _make_sc_gather():
    mesh = plsc.VectorSubcoreMesh(core_axis_name="c", subcore_axis_name="s")

    @functools.partial(
        pl.kernel,
        mesh=mesh,
        out_type=jax.ShapeDtypeStruct((N, C), jnp.float32),
        scratch_types=[
            pltpu.VMEM((NCH, CH), jnp.int32),
            pltpu.VMEM((NBUF, CH, C), jnp.float32),
        ]
        + [pltpu.SemaphoreType.DMA] * (2 * NBUF),
    )
    def _sc_gather(idx_hbm, table_hbm, out_hbm, idx_v, rows_v, *sems):
        sem_g, sem_o = sems[:NBUF], sems[NBUF:]
        wid = lax.axis_index("s") * NC + lax.axis_index("c")
        base = wid * ROWS_PER_W

        pltpu.sync_copy(idx_hbm.at[wid], idx_v)

        def start_gather(chunk, b):
            pltpu.async_copy(table_hbm.at[idx_v.at[chunk]], rows_v.at[b], sem_g[b])

        def wait_gather(b):
            pltpu.make_async_copy(
                out_hbm.at[pl.ds(0, CH)], rows_v.at[b], sem_g[b]
            ).wait()

        def start_out(chunk, b):
            off = base + chunk * CH
            pltpu.async_copy(rows_v.at[b], out_hbm.at[pl.ds(off, CH)], sem_o[b])

        def wait_out(b):
            pltpu.make_async_copy(
                rows_v.at[b], out_hbm.at[pl.ds(0, CH)], sem_o[b]
            ).wait()

        for b in range(NBUF):
            start_gather(b, b)

        def group(g, carry):
            i0 = g * NBUF
            for b in range(NBUF):
                chunk = i0 + b
                wait_gather(b)
                start_out(chunk, b)

                @pl.when(chunk + NBUF < NCH)
                def _():
                    wait_out(b)
                    start_gather(chunk + NBUF, b)

            return carry

        lax.fori_loop(0, NCH // NBUF, group, 0)
        for b in range(NBUF):
            wait_out(b)

    return _sc_gather


RB = 2048
G = N // RB


def _loss_body(x_ref, t_ref, out_ref):
    x = x_ref[...]
    t = t_ref[0, 0]
    m = jnp.max(x, axis=-1)
    s = jnp.sum(jnp.exp(x - m[:, None]), axis=-1)
    lse = m + jnp.log(s)
    cls = lax.broadcasted_iota(jnp.int32, (RB, C), 1)
    picked = jnp.sum(jnp.where(cls == t[:, None], x, 0.0), axis=-1)
    blk = jnp.sum(lse - picked)

    @pl.when(pl.program_id(0) == 0)
    def _():
        out_ref[0, 0] = 0.0

    out_ref[0, 0] += blk

    @pl.when(pl.program_id(0) == G - 1)
    def _():
        out_ref[0, 0] = out_ref[0, 0] / N


_tc_loss = pl.pallas_call(
    _loss_body,
    grid=(G,),
    in_specs=[
        pl.BlockSpec((RB, C), lambda i: (i, 0)),
        pl.BlockSpec((1, 1, RB), lambda i: (i, 0, 0)),
    ],
    out_specs=pl.BlockSpec((1, 1), lambda i: (0, 0), memory_space=pltpu.SMEM),
    out_shape=jax.ShapeDtypeStruct((1, 1), jnp.float32),
)


def kernel(idx, targets, table):
    idx_w = idx.reshape(NW, NCH, CH).astype(jnp.int32)
    logits = _make_sc_gather()(idx_w, table)
    tgt = targets.reshape(G, 1, RB).astype(jnp.int32)
    loss = _tc_loss(logits, tgt)[0, 0]
    return logits, loss


# speedup vs baseline: 4.8448x; 1.0569x over previous
import functools

import jax
import jax.numpy as jnp
from jax import lax
from jax.experimental import pallas as pl
from jax.experimental.pallas import tpu as pltpu
from jax.experimental.pallas import tpu_sc as plsc

N = 204800
C = 128
NC = 2
NS = 16
NW = NC * NS
ROWS_PER_W = N // NW
CH = 128
NCH = ROWS_PER_W // CH

NBUF = 5


@functools.cache
def _make_sc_gather():
    mesh = plsc.VectorSubcoreMesh(core_axis_name="c", subcore_axis_name="s")

    @functools.partial(
        pl.kernel,
        mesh=mesh,
        out_type=jax.ShapeDtypeStruct((N, C), jnp.float32),
        scratch_types=[
            pltpu.VMEM((NCH, CH), jnp.int32),
            pltpu.VMEM((NBUF, CH, C), jnp.float32),
        ]
        + [pltpu.SemaphoreType.DMA] * (2 * NBUF),
    )
    def _sc_gather(idx_hbm, table_hbm, out_hbm, idx_v, rows_v, *sems):
        sem_g, sem_o = sems[:NBUF], sems[NBUF:]
        wid = lax.axis_index("s") * NC + lax.axis_index("c")
        base = wid * ROWS_PER_W

        pltpu.sync_copy(idx_hbm.at[wid], idx_v)

        def start_gather(chunk, b):
            pltpu.async_copy(table_hbm.at[idx_v.at[chunk]], rows_v.at[b], sem_g[b])

        def wait_gather(b):
            pltpu.make_async_copy(
                out_hbm.at[pl.ds(0, CH)], rows_v.at[b], sem_g[b]
            ).wait()

        def start_out(chunk, b):
            off = base + chunk * CH
            pltpu.async_copy(rows_v.at[b], out_hbm.at[pl.ds(off, CH)], sem_o[b])

        def wait_out(b):
            pltpu.make_async_copy(
                rows_v.at[b], out_hbm.at[pl.ds(0, CH)], sem_o[b]
            ).wait()

        for b in range(NBUF):
            start_gather(b, b)

        def group(g, carry):
            i0 = g * NBUF
            for b in range(NBUF):
                chunk = i0 + b
                wait_gather(b)
                start_out(chunk, b)

                @pl.when(chunk + NBUF < NCH)
                def _():
                    wait_out(b)
                    start_gather(chunk + NBUF, b)

            return carry

        lax.fori_loop(0, NCH // NBUF, group, 0)
        for b in range(NBUF):
            wait_out(b)

    return _sc_gather


RB = 2048
G = N // RB


def _loss_body(x_ref, t_ref, out_ref):
    x = x_ref[...]
    t = t_ref[0, 0]
    k = jnp.max(x)
    e = jnp.exp(x - k)
    s = lax.dot_general(
        e,
        jnp.ones((C, 8), jnp.float32),
        (((1,), (0,)), ((), ())),
        preferred_element_type=jnp.float32,
    )[:, 0]
    lse_sum = RB * k + jnp.sum(jnp.log(s))
    cls = lax.broadcasted_iota(jnp.int32, (RB, C), 1)
    picked_sum = jnp.sum(jnp.where(cls == t[:, None], x, 0.0))
    blk = lse_sum - picked_sum

    @pl.when(pl.program_id(0) == 0)
    def _():
        out_ref[0, 0] = 0.0

    out_ref[0, 0] += blk

    @pl.when(pl.program_id(0) == G - 1)
    def _():
        out_ref[0, 0] = out_ref[0, 0] / N


_tc_loss = pl.pallas_call(
    _loss_body,
    grid=(G,),
    in_specs=[
        pl.BlockSpec((RB, C), lambda i: (i, 0)),
        pl.BlockSpec((1, 1, RB), lambda i: (i, 0, 0)),
    ],
    out_specs=pl.BlockSpec((1, 1), lambda i: (0, 0), memory_space=pltpu.SMEM),
    out_shape=jax.ShapeDtypeStruct((1, 1), jnp.float32),
)


def kernel(idx, targets, table):
    idx_w = idx.reshape(NW, NCH, CH).astype(jnp.int32)
    logits = _make_sc_gather()(idx_w, table)
    tgt = targets.reshape(G, 1, RB).astype(jnp.int32)
    loss = _tc_loss(logits, tgt)[0, 0]
    return logits, loss
